# Initial kernel scaffold; baseline (speedup 1.0000x reference)
import jax
import jax.numpy as jnp
from jax.experimental import pallas as pl


def kernel(x, edge_attr, edge_source, edge_target, Wf, bf, Ws, bs, gamma, beta):
    raise NotImplementedError("write your pallas kernel here")


# speedup vs baseline: 2.3539x; 2.3539x over previous
import functools

import jax
import jax.numpy as jnp
from jax import lax
from jax.experimental import pallas as pl
from jax.experimental.pallas import tpu as pltpu
from jax.experimental.pallas import tpu_sc as plsc

N = 10000
E = 320000
D = 128
D2 = 2 * D

NC = 2
NS = 16
NW = NC * NS
EPW = E // NW
CH = 80
NCHUNK = EPW // CH
NPAD = 10240
RPT = NPAD // NS
ZB = RPT // CH

@functools.lru_cache(maxsize=None)
def _mesh():
    return plsc.VectorSubcoreMesh(
        core_axis_name="c", subcore_axis_name="s",
        num_cores=NC, num_subcores=NS)


def _sigmoid(v):
    return 1.0 / (1.0 + jnp.exp(-v))


def _softplus(v):
    return jnp.maximum(v, 0.0) + jnp.log(1.0 + jnp.exp(-jnp.abs(v)))


_BN = 2000


def _pq_body(x_ref, ws_ref, wd_ref, b_ref, p_ref, q_ref):
    xb = x_ref[...]
    p_ref[...] = jnp.dot(xb, ws_ref[...],
                         preferred_element_type=jnp.float32) + b_ref[...]
    q_ref[...] = jnp.dot(xb, wd_ref[...], preferred_element_type=jnp.float32)


def _pq_call(x, wsrc, wdst, b2):
    return pl.pallas_call(
        _pq_body,
        grid=(N // _BN,),
        in_specs=[
            pl.BlockSpec((_BN, D), lambda i: (i, 0)),
            pl.BlockSpec((D, D2), lambda i: (0, 0)),
            pl.BlockSpec((D, D2), lambda i: (0, 0)),
            pl.BlockSpec((1, D2), lambda i: (0, 0)),
        ],
        out_specs=[
            pl.BlockSpec((_BN, D2), lambda i: (i, 0)),
            pl.BlockSpec((_BN, D2), lambda i: (i, 0)),
        ],
        out_shape=[
            jax.ShapeDtypeStruct((N, D2), jnp.float32),
            jax.ShapeDtypeStruct((N, D2), jnp.float32),
        ],
    )(x, wsrc, wdst, b2)


@functools.lru_cache(maxsize=None)
def _gather_add_kernel():
    return pl.kernel(
        _gather_add_body,
        out_type=jax.ShapeDtypeStruct((E, D2), jnp.float32),
        mesh=_mesh(),
        scratch_types=[
            pltpu.VMEM((CH,), jnp.int32),
            pltpu.VMEM((CH,), jnp.int32),
            pltpu.VMEM((CH, D2), jnp.float32),
            pltpu.VMEM((CH, D2), jnp.float32),
            pltpu.SemaphoreType.DMA,
            pltpu.SemaphoreType.DMA,
        ],
    )


def _gather_add_body(p_hbm, q_hbm, src_hbm, dst_hbm, g_hbm,
                     sidx, didx, pbuf, qbuf, sem1, sem2):
    wid = lax.axis_index("s") * NC + lax.axis_index("c")
    base = wid * EPW

    def chunk(i, carry):
        off = base + i * CH
        pltpu.sync_copy(src_hbm.at[pl.ds(off, CH)], sidx)
        pltpu.sync_copy(dst_hbm.at[pl.ds(off, CH)], didx)
        cp1 = pltpu.async_copy(p_hbm.at[sidx], pbuf, sem1)
        cp2 = pltpu.async_copy(q_hbm.at[didx], qbuf, sem2)
        cp1.wait()
        cp2.wait()

        def row(r, c2):
            for j in range(D2 // 16):
                sl = pl.ds(j * 16, 16)
                pbuf[r, sl] = pbuf[r, sl] + qbuf[r, sl]
            return c2

        lax.fori_loop(0, CH, row, 0)
        pltpu.sync_copy(pbuf, g_hbm.at[pl.ds(off, CH)])
        return carry

    lax.fori_loop(0, NCHUNK, chunk, 0)


_BE = 2000


def _edge_body(ea_ref, g_ref, we_ref, m_ref):
    logits = jnp.dot(ea_ref[...], we_ref[...],
                     preferred_element_type=jnp.float32) + g_ref[...]
    f = _sigmoid(logits[:, :D])
    s = _softplus(logits[:, D:])
    m_ref[...] = f * s


def _edge_call(ea, g, wea):
    return pl.pallas_call(
        _edge_body,
        grid=(E // _BE,),
        in_specs=[
            pl.BlockSpec((_BE, D), lambda i: (i, 0)),
            pl.BlockSpec((_BE, D2), lambda i: (i, 0)),
            pl.BlockSpec((D, D2), lambda i: (0, 0)),
        ],
        out_specs=pl.BlockSpec((_BE, D), lambda i: (i, 0)),
        out_shape=jax.ShapeDtypeStruct((E, D), jnp.float32),
    )(ea, g, wea)


@functools.lru_cache(maxsize=None)
def _scatter_add_kernel():
    return pl.kernel(
        _scatter_add_body,
        out_type=jax.ShapeDtypeStruct((NC * NPAD, D), jnp.float32),
        mesh=_mesh(),
        scratch_types=[
            pltpu.VMEM((CH,), jnp.int32),
            pltpu.VMEM((CH, D), jnp.float32),
            pltpu.VMEM_SHARED((NPAD, D), jnp.float32),
        ],
    )


def _scatter_add_body(m_hbm, src_hbm, out_hbm, idxb, mbuf, acc_sh):
    cid = lax.axis_index("c")
    sid = lax.axis_index("s")
    wid = sid * NC + cid

    def zrow(r, c):
        for j in range(D // 16):
            mbuf[r, pl.ds(j * 16, 16)] = jnp.zeros((16,), jnp.float32)
        return c

    lax.fori_loop(0, CH, zrow, 0)
    for t in range(ZB):
        pltpu.sync_copy(mbuf, acc_sh.at[pl.ds(sid * RPT + t * CH, CH)])
    plsc.subcore_barrier()

    base = wid * EPW

    def chunk(i, carry):
        off = base + i * CH
        pltpu.sync_copy(m_hbm.at[pl.ds(off, CH)], mbuf)
        pltpu.sync_copy(src_hbm.at[pl.ds(off, CH)], idxb)
        pltpu.sync_copy(mbuf, acc_sh.at[idxb], add=True)
        return carry

    lax.fori_loop(0, NCHUNK, chunk, 0)
    plsc.subcore_barrier()

    for t in range(ZB):
        row0 = sid * RPT + t * CH
        pltpu.sync_copy(acc_sh.at[pl.ds(row0, CH)],
                        out_hbm.at[pl.ds(cid * NPAD + row0, CH)])


def _final_body(mp_ref, x_ref, gam_ref, bet_ref, o_ref):
    msg = mp_ref[0:N, :] + mp_ref[NPAD:NPAD + N, :]
    mean = jnp.mean(msg, axis=0, keepdims=True)
    var = jnp.mean((msg - mean) ** 2, axis=0, keepdims=True)
    norm = (msg - mean) / jnp.sqrt(var + 1e-5) * gam_ref[...] + bet_ref[...]
    o_ref[...] = _softplus(x_ref[...] + norm)


def _final_call(mp, x, gamma, beta):
    return pl.pallas_call(
        _final_body,
        out_shape=jax.ShapeDtypeStruct((N, D), jnp.float32),
    )(mp, x, gamma, beta)


def kernel(x, edge_attr, edge_source, edge_target, Wf, bf, Ws, bs, gamma, beta):
    src = edge_source.astype(jnp.int32)
    dst = edge_target.astype(jnp.int32)
    b2 = jnp.concatenate([bf, bs]).reshape(1, D2)
    wsrc = jnp.concatenate([Wf[:, :D].T, Ws[:, :D].T], axis=1)
    wdst = jnp.concatenate([Wf[:, D:2 * D].T, Ws[:, D:2 * D].T], axis=1)
    wea = jnp.concatenate([Wf[:, 2 * D:].T, Ws[:, 2 * D:].T], axis=1)

    p, q = _pq_call(x, wsrc, wdst, b2)
    g = _gather_add_kernel()(p, q, src, dst)
    m = _edge_call(edge_attr, g, wea)
    mp = _scatter_add_kernel()(m, src)
    return _final_call(mp, x, gamma.reshape(1, D), beta.reshape(1, D))


# speedup vs baseline: 3.6935x; 1.5691x over previous
import functools

import jax
import jax.numpy as jnp
from jax import lax
from jax.experimental import pallas as pl
from jax.experimental.pallas import tpu as pltpu
from jax.experimental.pallas import tpu_sc as plsc

N = 10000
E = 320000
D = 128
D2 = 2 * D

NC = 2
NS = 16
NW = NC * NS
EPW = E // NW
CH = 80
NCHUNK = EPW // CH
NPAD = 10240
RPT = NPAD // NS
ZB = RPT // CH


@functools.lru_cache(maxsize=None)
def _mesh():
    return plsc.VectorSubcoreMesh(
        core_axis_name="c", subcore_axis_name="s",
        num_cores=NC, num_subcores=NS)


def _sigmoid(v):
    return 1.0 / (1.0 + jnp.exp(-v))


def _softplus(v):
    return jnp.maximum(v, 0.0) + jnp.log(1.0 + jnp.exp(-jnp.abs(v)))


def _pack2(v):
    lo = lax.bitcast_convert_type(v[:, :D].astype(jnp.bfloat16), jnp.uint16)
    hi = lax.bitcast_convert_type(v[:, D:].astype(jnp.bfloat16), jnp.uint16)
    w = lo.astype(jnp.uint32) | (hi.astype(jnp.uint32) << 16)
    return lax.bitcast_convert_type(w, jnp.int32)


_BN = 2000


def _pq_body(x_ref, ws_ref, wd_ref, b_ref, p_ref, q_ref):
    xb = x_ref[...]
    p = jnp.dot(xb, ws_ref[...],
                preferred_element_type=jnp.float32) + b_ref[...]
    q = jnp.dot(xb, wd_ref[...], preferred_element_type=jnp.float32)
    p_ref[...] = _pack2(p)
    q_ref[...] = _pack2(q)


def _pq_call(x, wsrc, wdst, b2):
    return pl.pallas_call(
        _pq_body,
        grid=(N // _BN,),
        in_specs=[
            pl.BlockSpec((_BN, D), lambda i: (i, 0)),
            pl.BlockSpec((D, D2), lambda i: (0, 0)),
            pl.BlockSpec((D, D2), lambda i: (0, 0)),
            pl.BlockSpec((1, D2), lambda i: (0, 0)),
        ],
        out_specs=[
            pl.BlockSpec((_BN, D), lambda i: (i, 0)),
            pl.BlockSpec((_BN, D), lambda i: (i, 0)),
        ],
        out_shape=[
            jax.ShapeDtypeStruct((N, D), jnp.int32),
            jax.ShapeDtypeStruct((N, D), jnp.int32),
        ],
    )(x, wsrc, wdst, b2)


_RING = 4


@functools.lru_cache(maxsize=None)
def _gather_kernel():
    return pl.kernel(
        _gather_body,
        out_type=[
            jax.ShapeDtypeStruct((E, D), jnp.int32),
            jax.ShapeDtypeStruct((E, D), jnp.int32),
        ],
        mesh=_mesh(),
        scratch_types=(
            [pltpu.VMEM((EPW,), jnp.int32)] * 2
            + [pltpu.VMEM((CH, D), jnp.int32)] * (2 * _RING)
            + [pltpu.SemaphoreType.DMA] * (2 * _RING)
        ),
    )


def _gather_body(p_hbm, q_hbm, src_hbm, dst_hbm, gs_hbm, gd_hbm, *bufs):
    sidx, didx = bufs[0], bufs[1]
    pbs = bufs[2:2 + _RING]
    qbs = bufs[2 + _RING:2 + 2 * _RING]
    gsems = bufs[2 + 2 * _RING:2 + 3 * _RING]
    wsems = bufs[2 + 3 * _RING:2 + 4 * _RING]

    wid = lax.axis_index("s") * NC + lax.axis_index("c")
    base = wid * EPW
    pltpu.sync_copy(src_hbm.at[pl.ds(base, EPW)], sidx)
    pltpu.sync_copy(dst_hbm.at[pl.ds(base, EPW)], didx)

    def issue_gather(c, b):
        isl = pl.ds(c * CH, CH)
        pltpu.async_copy(p_hbm.at[sidx.at[isl]], pbs[b], gsems[b])
        pltpu.async_copy(q_hbm.at[didx.at[isl]], qbs[b], gsems[b])

    def wait_gather(b):
        pltpu.make_async_copy(p_hbm.at[sidx.at[pl.ds(0, CH)]], pbs[b],
                              gsems[b]).wait()
        pltpu.make_async_copy(q_hbm.at[didx.at[pl.ds(0, CH)]], qbs[b],
                              gsems[b]).wait()

    def issue_wb(c, b):
        osl = pl.ds(base + c * CH, CH)
        pltpu.async_copy(pbs[b], gs_hbm.at[osl], wsems[b])
        pltpu.async_copy(qbs[b], gd_hbm.at[osl], wsems[b])

    def wait_wb(b):
        pltpu.make_async_copy(pbs[b], gs_hbm.at[pl.ds(0, CH)],
                              wsems[b]).wait()
        pltpu.make_async_copy(qbs[b], gd_hbm.at[pl.ds(0, CH)],
                              wsems[b]).wait()

    for b in range(_RING - 1):
        issue_gather(b, b)

    def quad(i4, carry):
        for b in range(_RING):
            c = i4 * _RING + b
            tb = (b + 3) % _RING

            @pl.when(c >= 1)
            def _():
                wait_wb(tb)

            @pl.when(c + (_RING - 1) < NCHUNK)
            def _():
                issue_gather(c + (_RING - 1), tb)

            wait_gather(b)
            issue_wb(c, b)
        return carry

    lax.fori_loop(0, NCHUNK // _RING, quad, 0)

    c_tail = (NCHUNK // _RING) * _RING
    wait_wb(_RING - 1)
    wait_gather(0)
    issue_wb(c_tail, 0)
    wait_wb(0)


_BE = 2000


def _edge_body(ea_ref, gs_ref, gd_ref, we_ref, m_ref):
    ll = jnp.dot(ea_ref[...], we_ref[...], preferred_element_type=jnp.float32)
    gs = gs_ref[...]
    gd = gd_ref[...]
    lf = (lax.bitcast_convert_type(gs << 16, jnp.float32)
          + lax.bitcast_convert_type(gd << 16, jnp.float32))
    ls = (lax.bitcast_convert_type(gs & jnp.int32(-65536), jnp.float32)
          + lax.bitcast_convert_type(gd & jnp.int32(-65536), jnp.float32))
    f = _sigmoid(ll[:, :D] + lf)
    s = _softplus(ll[:, D:] + ls)
    m_ref[...] = f * s


def _edge_call(ea, gs, gd, wea):
    return pl.pallas_call(
        _edge_body,
        grid=(E // _BE,),
        in_specs=[
            pl.BlockSpec((_BE, D), lambda i: (i, 0)),
            pl.BlockSpec((_BE, D), lambda i: (i, 0)),
            pl.BlockSpec((_BE, D), lambda i: (i, 0)),
            pl.BlockSpec((D, D2), lambda i: (0, 0)),
        ],
        out_specs=pl.BlockSpec((_BE, D), lambda i: (i, 0)),
        out_shape=jax.ShapeDtypeStruct((E, D), jnp.float32),
    )(ea, gs, gd, wea)


@functools.lru_cache(maxsize=None)
def _scatter_add_kernel():
    return pl.kernel(
        _scatter_add_body,
        out_type=jax.ShapeDtypeStruct((NC * NPAD, D), jnp.float32),
        mesh=_mesh(),
        scratch_types=[
            pltpu.VMEM((CH,), jnp.int32),
            pltpu.VMEM((CH, D), jnp.float32),
            pltpu.VMEM_SHARED((NPAD, D), jnp.float32),
        ],
    )


def _scatter_add_body(m_hbm, src_hbm, out_hbm, idxb, mbuf, acc_sh):
    cid = lax.axis_index("c")
    sid = lax.axis_index("s")
    wid = sid * NC + cid

    def zrow(r, c):
        for j in range(D // 16):
            mbuf[r, pl.ds(j * 16, 16)] = jnp.zeros((16,), jnp.float32)
        return c

    lax.fori_loop(0, CH, zrow, 0)
    for t in range(ZB):
        pltpu.sync_copy(mbuf, acc_sh.at[pl.ds(sid * RPT + t * CH, CH)])
    plsc.subcore_barrier()

    base = wid * EPW

    def chunk(i, carry):
        off = base + i * CH
        pltpu.sync_copy(m_hbm.at[pl.ds(off, CH)], mbuf)
        pltpu.sync_copy(src_hbm.at[pl.ds(off, CH)], idxb)
        pltpu.sync_copy(mbuf, acc_sh.at[idxb], add=True)
        return carry

    lax.fori_loop(0, NCHUNK, chunk, 0)
    plsc.subcore_barrier()

    for t in range(ZB):
        row0 = sid * RPT + t * CH
        pltpu.sync_copy(acc_sh.at[pl.ds(row0, CH)],
                        out_hbm.at[pl.ds(cid * NPAD + row0, CH)])


def _final_body(mp_ref, x_ref, gam_ref, bet_ref, o_ref):
    msg = mp_ref[0:N, :] + mp_ref[NPAD:NPAD + N, :]
    mean = jnp.mean(msg, axis=0, keepdims=True)
    var = jnp.mean((msg - mean) ** 2, axis=0, keepdims=True)
    norm = (msg - mean) / jnp.sqrt(var + 1e-5) * gam_ref[...] + bet_ref[...]
    o_ref[...] = _softplus(x_ref[...] + norm)


def _final_call(mp, x, gamma, beta):
    return pl.pallas_call(
        _final_body,
        out_shape=jax.ShapeDtypeStruct((N, D), jnp.float32),
    )(mp, x, gamma, beta)


def kernel(x, edge_attr, edge_source, edge_target, Wf, bf, Ws, bs, gamma, beta):
    src = edge_source.astype(jnp.int32)
    dst = edge_target.astype(jnp.int32)
    b2 = jnp.concatenate([bf, bs]).reshape(1, D2)
    wsrc = jnp.concatenate([Wf[:, :D].T, Ws[:, :D].T], axis=1)
    wdst = jnp.concatenate([Wf[:, D:2 * D].T, Ws[:, D:2 * D].T], axis=1)
    wea = jnp.concatenate([Wf[:, 2 * D:].T, Ws[:, 2 * D:].T], axis=1)

    p, q = _pq_call(x, wsrc, wdst, b2)
    gs, gd = _gather_kernel()(p, q, src, dst)
    m = _edge_call(edge_attr, gs, gd, wea)
    mp = _scatter_add_kernel()(m, src)
    return _final_call(mp, x, gamma.reshape(1, D), beta.reshape(1, D))


# speedup vs baseline: 4.6367x; 1.2554x over previous
import functools

import jax
import jax.numpy as jnp
from jax import lax
from jax.experimental import pallas as pl
from jax.experimental.pallas import tpu as pltpu
from jax.experimental.pallas import tpu_sc as plsc

N = 10000
E = 320000
D = 128
D2 = 2 * D

NC = 2
NS = 16
NW = NC * NS
EPW = E // NW
CH = 80
NCHUNK = EPW // CH
NPAD = 10240
RPT = NPAD // NS
ZB = RPT // CH


@functools.lru_cache(maxsize=None)
def _mesh():
    return plsc.VectorSubcoreMesh(
        core_axis_name="c", subcore_axis_name="s",
        num_cores=NC, num_subcores=NS)


def _sigmoid(v):
    return 1.0 / (1.0 + jnp.exp(-v))


def _softplus(v):
    return jnp.maximum(v, 0.0) + jnp.log(1.0 + jnp.exp(-jnp.abs(v)))


def _pack2(v):
    lo = lax.bitcast_convert_type(v[:, :D].astype(jnp.bfloat16), jnp.uint16)
    hi = lax.bitcast_convert_type(v[:, D:].astype(jnp.bfloat16), jnp.uint16)
    w = lo.astype(jnp.uint32) | (hi.astype(jnp.uint32) << 16)
    return lax.bitcast_convert_type(w, jnp.int32)


_BN = 2000


def _pq_body(x_ref, ws_ref, wd_ref, b_ref, p_ref, q_ref):
    xb = x_ref[...]
    p = jnp.dot(xb, ws_ref[...],
                preferred_element_type=jnp.float32) + b_ref[...]
    q = jnp.dot(xb, wd_ref[...], preferred_element_type=jnp.float32)
    p_ref[...] = _pack2(p)
    q_ref[...] = _pack2(q)


def _pq_call(x, wsrc, wdst, b2):
    return pl.pallas_call(
        _pq_body,
        grid=(N // _BN,),
        in_specs=[
            pl.BlockSpec((_BN, D), lambda i: (i, 0)),
            pl.BlockSpec((D, D2), lambda i: (0, 0)),
            pl.BlockSpec((D, D2), lambda i: (0, 0)),
            pl.BlockSpec((1, D2), lambda i: (0, 0)),
        ],
        out_specs=[
            pl.BlockSpec((_BN, D), lambda i: (i, 0)),
            pl.BlockSpec((_BN, D), lambda i: (i, 0)),
        ],
        out_shape=[
            jax.ShapeDtypeStruct((N, D), jnp.int32),
            jax.ShapeDtypeStruct((N, D), jnp.int32),
        ],
    )(x, wsrc, wdst, b2)


_RING = 4


@functools.lru_cache(maxsize=None)
def _gather_kernel():
    return pl.kernel(
        _gather_body,
        out_type=[
            jax.ShapeDtypeStruct((E, D), jnp.int32),
            jax.ShapeDtypeStruct((E, D), jnp.int32),
        ],
        mesh=_mesh(),
        scratch_types=(
            [pltpu.VMEM((EPW,), jnp.int32)] * 2
            + [pltpu.VMEM((CH, D), jnp.int32)] * (2 * _RING)
            + [pltpu.SemaphoreType.DMA] * (2 * _RING)
        ),
    )


def _gather_body(p_hbm, q_hbm, src_hbm, dst_hbm, gs_hbm, gd_hbm, *bufs):
    sidx, didx = bufs[0], bufs[1]
    pbs = bufs[2:2 + _RING]
    qbs = bufs[2 + _RING:2 + 2 * _RING]
    gsems = bufs[2 + 2 * _RING:2 + 3 * _RING]
    wsems = bufs[2 + 3 * _RING:2 + 4 * _RING]

    wid = lax.axis_index("s") * NC + lax.axis_index("c")
    base = wid * EPW
    pltpu.sync_copy(src_hbm.at[pl.ds(base, EPW)], sidx)
    pltpu.sync_copy(dst_hbm.at[pl.ds(base, EPW)], didx)

    def issue_gather(c, b):
        isl = pl.ds(c * CH, CH)
        pltpu.async_copy(p_hbm.at[sidx.at[isl]], pbs[b], gsems[b])
        pltpu.async_copy(q_hbm.at[didx.at[isl]], qbs[b], gsems[b])

    def wait_gather(b):
        pltpu.make_async_copy(p_hbm.at[sidx.at[pl.ds(0, CH)]], pbs[b],
                              gsems[b]).wait()
        pltpu.make_async_copy(q_hbm.at[didx.at[pl.ds(0, CH)]], qbs[b],
                              gsems[b]).wait()

    def issue_wb(c, b):
        osl = pl.ds(base + c * CH, CH)
        pltpu.async_copy(pbs[b], gs_hbm.at[osl], wsems[b])
        pltpu.async_copy(qbs[b], gd_hbm.at[osl], wsems[b])

    def wait_wb(b):
        pltpu.make_async_copy(pbs[b], gs_hbm.at[pl.ds(0, CH)],
                              wsems[b]).wait()
        pltpu.make_async_copy(qbs[b], gd_hbm.at[pl.ds(0, CH)],
                              wsems[b]).wait()

    for b in range(_RING - 1):
        issue_gather(b, b)

    def quad(i4, carry):
        for b in range(_RING):
            c = i4 * _RING + b
            tb = (b + 3) % _RING

            @pl.when(c >= 1)
            def _():
                wait_wb(tb)

            @pl.when(c + (_RING - 1) < NCHUNK)
            def _():
                issue_gather(c + (_RING - 1), tb)

            wait_gather(b)
            issue_wb(c, b)
        return carry

    lax.fori_loop(0, NCHUNK // _RING, quad, 0)

    c_tail = (NCHUNK // _RING) * _RING
    wait_wb(_RING - 1)
    wait_gather(0)
    issue_wb(c_tail, 0)
    wait_wb(0)


_BE = 2000


def _edge_body(ea_ref, gs_ref, gd_ref, we_ref, m_ref):
    ll = jnp.dot(ea_ref[...], we_ref[...], preferred_element_type=jnp.float32)
    gs = gs_ref[...]
    gd = gd_ref[...]
    lf = (lax.bitcast_convert_type(gs << 16, jnp.float32)
          + lax.bitcast_convert_type(gd << 16, jnp.float32))
    ls = (lax.bitcast_convert_type(gs & jnp.int32(-65536), jnp.float32)
          + lax.bitcast_convert_type(gd & jnp.int32(-65536), jnp.float32))
    f = _sigmoid(ll[:, :D] + lf)
    s = _softplus(ll[:, D:] + ls)
    m_ref[...] = f * s


def _edge_call(ea, gs, gd, wea):
    return pl.pallas_call(
        _edge_body,
        grid=(E // _BE,),
        in_specs=[
            pl.BlockSpec((_BE, D), lambda i: (i, 0)),
            pl.BlockSpec((_BE, D), lambda i: (i, 0)),
            pl.BlockSpec((_BE, D), lambda i: (i, 0)),
            pl.BlockSpec((D, D2), lambda i: (0, 0)),
        ],
        out_specs=pl.BlockSpec((_BE, D), lambda i: (i, 0)),
        out_shape=jax.ShapeDtypeStruct((E, D), jnp.float32),
    )(ea, gs, gd, wea)


@functools.lru_cache(maxsize=None)
def _scatter_add_kernel():
    return pl.kernel(
        _scatter_add_body,
        out_type=jax.ShapeDtypeStruct((NC * NPAD, D), jnp.float32),
        mesh=_mesh(),
        scratch_types=(
            [pltpu.VMEM((CH,), jnp.int32)] * _RING
            + [pltpu.VMEM((CH, D), jnp.float32)] * _RING
            + [pltpu.VMEM_SHARED((NPAD, D), jnp.float32)]
            + [pltpu.SemaphoreType.DMA] * (2 * _RING)
        ),
    )


def _scatter_add_body(m_hbm, src_hbm, out_hbm, *bufs):
    idxbs = bufs[:_RING]
    mbufs = bufs[_RING:2 * _RING]
    acc_sh = bufs[2 * _RING]
    lsems = bufs[2 * _RING + 1:3 * _RING + 1]
    ssems = bufs[3 * _RING + 1:4 * _RING + 1]

    cid = lax.axis_index("c")
    sid = lax.axis_index("s")
    wid = sid * NC + cid
    base = wid * EPW

    def zrow(r, c):
        for j in range(D // 16):
            mbufs[0][r, pl.ds(j * 16, 16)] = jnp.zeros((16,), jnp.float32)
        return c

    lax.fori_loop(0, CH, zrow, 0)
    for t in range(ZB):
        pltpu.sync_copy(mbufs[0], acc_sh.at[pl.ds(sid * RPT + t * CH, CH)])
    plsc.subcore_barrier()

    def issue_load(c, b):
        sl = pl.ds(base + c * CH, CH)
        pltpu.async_copy(m_hbm.at[sl], mbufs[b], lsems[b])
        pltpu.async_copy(src_hbm.at[sl], idxbs[b], lsems[b])

    def wait_load(b):
        pltpu.make_async_copy(m_hbm.at[pl.ds(0, CH)], mbufs[b],
                              lsems[b]).wait()
        pltpu.make_async_copy(src_hbm.at[pl.ds(0, CH)], idxbs[b],
                              lsems[b]).wait()

    def issue_scatter(b):
        pltpu.async_copy(mbufs[b], acc_sh.at[idxbs[b]], ssems[b], add=True)

    def wait_scatter(b):
        pltpu.make_async_copy(mbufs[b], acc_sh.at[idxbs[b]], ssems[b]).wait()

    for b in range(_RING - 1):
        issue_load(b, b)

    def quad(i4, carry):
        for b in range(_RING):
            c = i4 * _RING + b
            tb = (b + 3) % _RING

            @pl.when(c >= 1)
            def _():
                wait_scatter(tb)

            @pl.when(c + (_RING - 1) < NCHUNK)
            def _():
                issue_load(c + (_RING - 1), tb)

            wait_load(b)
            issue_scatter(b)
        return carry

    lax.fori_loop(0, NCHUNK // _RING, quad, 0)

    wait_scatter(_RING - 1)
    wait_load(0)
    issue_scatter(0)
    wait_scatter(0)
    plsc.subcore_barrier()

    for t in range(ZB):
        row0 = sid * RPT + t * CH
        pltpu.sync_copy(acc_sh.at[pl.ds(row0, CH)],
                        out_hbm.at[pl.ds(cid * NPAD + row0, CH)])


def _final_body(mp_ref, x_ref, gam_ref, bet_ref, o_ref):
    msg = mp_ref[0:N, :] + mp_ref[NPAD:NPAD + N, :]
    mean = jnp.mean(msg, axis=0, keepdims=True)
    var = jnp.mean((msg - mean) ** 2, axis=0, keepdims=True)
    norm = (msg - mean) / jnp.sqrt(var + 1e-5) * gam_ref[...] + bet_ref[...]
    o_ref[...] = _softplus(x_ref[...] + norm)


def _final_call(mp, x, gamma, beta):
    return pl.pallas_call(
        _final_body,
        out_shape=jax.ShapeDtypeStruct((N, D), jnp.float32),
    )(mp, x, gamma, beta)


def kernel(x, edge_attr, edge_source, edge_target, Wf, bf, Ws, bs, gamma, beta):
    src = edge_source.astype(jnp.int32)
    dst = edge_target.astype(jnp.int32)
    b2 = jnp.concatenate([bf, bs]).reshape(1, D2)
    wsrc = jnp.concatenate([Wf[:, :D].T, Ws[:, :D].T], axis=1)
    wdst = jnp.concatenate([Wf[:, D:2 * D].T, Ws[:, D:2 * D].T], axis=1)
    wea = jnp.concatenate([Wf[:, 2 * D:].T, Ws[:, 2 * D:].T], axis=1)

    p, q = _pq_call(x, wsrc, wdst, b2)
    gs, gd = _gather_kernel()(p, q, src, dst)
    m = _edge_call(edge_attr, gs, gd, wea)
    mp = _scatter_add_kernel()(m, src)
    return _final_call(mp, x, gamma.reshape(1, D), beta.reshape(1, D))


# speedup vs baseline: 4.9070x; 1.0583x over previous
import functools

import jax
import jax.numpy as jnp
from jax import lax
from jax.experimental import pallas as pl
from jax.experimental.pallas import tpu as pltpu
from jax.experimental.pallas import tpu_sc as plsc

N = 10000
E = 320000
D = 128
D2 = 2 * D

NC = 2
NS = 16
NW = NC * NS
NH = 2
EH = E // NH
EPW = EH // NW
CH = 40
NCHUNK = EPW // CH
_RING = 4
NPAD = 10240
RPT = NPAD // NS
ZCH = 80
ZB = RPT // ZCH


@functools.lru_cache(maxsize=None)
def _mesh():
    return plsc.VectorSubcoreMesh(
        core_axis_name="c", subcore_axis_name="s",
        num_cores=NC, num_subcores=NS)


def _sigmoid(v):
    return 1.0 / (1.0 + jnp.exp(-v))


def _softplus(v):
    return jnp.maximum(v, 0.0) + jnp.log(1.0 + jnp.exp(-jnp.abs(v)))


def _pack2(v):
    lo = lax.bitcast_convert_type(v[:, :D].astype(jnp.bfloat16), jnp.uint16)
    hi = lax.bitcast_convert_type(v[:, D:].astype(jnp.bfloat16), jnp.uint16)
    w = lo.astype(jnp.uint32) | (hi.astype(jnp.uint32) << 16)
    return lax.bitcast_convert_type(w, jnp.int32)


_BN = 2000


def _pq_body(x_ref, ws_ref, wd_ref, b_ref, p_ref, q_ref):
    xb = x_ref[...]
    p = jnp.dot(xb, ws_ref[...],
                preferred_element_type=jnp.float32) + b_ref[...]
    q = jnp.dot(xb, wd_ref[...], preferred_element_type=jnp.float32)
    p_ref[...] = _pack2(p)
    q_ref[...] = _pack2(q)


def _pq_call(x, wsrc, wdst, b2):
    return pl.pallas_call(
        _pq_body,
        grid=(N // _BN,),
        in_specs=[
            pl.BlockSpec((_BN, D), lambda i: (i, 0)),
            pl.BlockSpec((D, D2), lambda i: (0, 0)),
            pl.BlockSpec((D, D2), lambda i: (0, 0)),
            pl.BlockSpec((1, D2), lambda i: (0, 0)),
        ],
        out_specs=[
            pl.BlockSpec((_BN, D), lambda i: (i, 0)),
            pl.BlockSpec((_BN, D), lambda i: (i, 0)),
        ],
        out_shape=[
            jax.ShapeDtypeStruct((N, D), jnp.int32),
            jax.ShapeDtypeStruct((N, D), jnp.int32),
        ],
    )(x, wsrc, wdst, b2)


def _make_gather_body(e0):
    def body(p_hbm, q_hbm, src_hbm, dst_hbm, gs_hbm, gd_hbm, *bufs):
        sidx, didx = bufs[0], bufs[1]
        pbs = bufs[2:2 + _RING]
        qbs = bufs[2 + _RING:2 + 2 * _RING]
        gsems = bufs[2 + 2 * _RING:2 + 3 * _RING]
        wsems = bufs[2 + 3 * _RING:2 + 4 * _RING]

        wid = lax.axis_index("s") * NC + lax.axis_index("c")
        lbase = wid * EPW
        gbase = e0 + lbase
        pltpu.sync_copy(src_hbm.at[pl.ds(gbase, EPW)], sidx)
        pltpu.sync_copy(dst_hbm.at[pl.ds(gbase, EPW)], didx)

        def issue_gather(c, b):
            isl = pl.ds(c * CH, CH)
            pltpu.async_copy(p_hbm.at[sidx.at[isl]], pbs[b], gsems[b])
            pltpu.async_copy(q_hbm.at[didx.at[isl]], qbs[b], gsems[b])

        def wait_gather(b):
            pltpu.make_async_copy(p_hbm.at[sidx.at[pl.ds(0, CH)]], pbs[b],
                                  gsems[b]).wait()
            pltpu.make_async_copy(q_hbm.at[didx.at[pl.ds(0, CH)]], qbs[b],
                                  gsems[b]).wait()

        def issue_wb(c, b):
            osl = pl.ds(lbase + c * CH, CH)
            pltpu.async_copy(pbs[b], gs_hbm.at[osl], wsems[b])
            pltpu.async_copy(qbs[b], gd_hbm.at[osl], wsems[b])

        def wait_wb(b):
            pltpu.make_async_copy(pbs[b], gs_hbm.at[pl.ds(0, CH)],
                                  wsems[b]).wait()
            pltpu.make_async_copy(qbs[b], gd_hbm.at[pl.ds(0, CH)],
                                  wsems[b]).wait()

        for b in range(_RING - 1):
            issue_gather(b, b)

        def quad(i4, carry):
            for b in range(_RING):
                c = i4 * _RING + b
                tb = (b + 3) % _RING

                @pl.when(c >= 1)
                def _():
                    wait_wb(tb)

                @pl.when(c + (_RING - 1) < NCHUNK)
                def _():
                    issue_gather(c + (_RING - 1), tb)

                wait_gather(b)
                issue_wb(c, b)
            return carry

        lax.fori_loop(0, NCHUNK // _RING, quad, 0)

        c_tail = (NCHUNK // _RING) * _RING
        wait_wb(_RING - 1)
        wait_gather(0)
        issue_wb(c_tail, 0)
        wait_wb(0)

    return body


@functools.lru_cache(maxsize=None)
def _gather_kernel(h):
    return pl.kernel(
        _make_gather_body(h * EH),
        out_type=[
            jax.ShapeDtypeStruct((EH, D), jnp.int32),
            jax.ShapeDtypeStruct((EH, D), jnp.int32),
        ],
        mesh=_mesh(),
        scratch_types=(
            [pltpu.VMEM((EPW,), jnp.int32)] * 2
            + [pltpu.VMEM((CH, D), jnp.int32)] * (2 * _RING)
            + [pltpu.SemaphoreType.DMA] * (2 * _RING)
        ),
    )


_BE = 2000


def _edge_body(ea_ref, gs_ref, gd_ref, we_ref, m_ref):
    ll = jnp.dot(ea_ref[...], we_ref[...], preferred_element_type=jnp.float32)
    gs = gs_ref[...]
    gd = gd_ref[...]
    lf = (lax.bitcast_convert_type(gs << 16, jnp.float32)
          + lax.bitcast_convert_type(gd << 16, jnp.float32))
    ls = (lax.bitcast_convert_type(gs & jnp.int32(-65536), jnp.float32)
          + lax.bitcast_convert_type(gd & jnp.int32(-65536), jnp.float32))
    f = _sigmoid(ll[:, :D] + lf)
    s = _softplus(ll[:, D:] + ls)
    m_ref[...] = f * s


def _edge_call(ea, gs, gd, wea, h):
    hoff = h * (EH // _BE)
    return pl.pallas_call(
        _edge_body,
        grid=(EH // _BE,),
        in_specs=[
            pl.BlockSpec((_BE, D), lambda i: (i + hoff, 0)),
            pl.BlockSpec((_BE, D), lambda i: (i, 0)),
            pl.BlockSpec((_BE, D), lambda i: (i, 0)),
            pl.BlockSpec((D, D2), lambda i: (0, 0)),
        ],
        out_specs=pl.BlockSpec((_BE, D), lambda i: (i, 0)),
        out_shape=jax.ShapeDtypeStruct((EH, D), jnp.float32),
    )(ea, gs, gd, wea)


def _make_scatter_body(e0):
    def body(m_hbm, src_hbm, out_hbm, *bufs):
        idxbs = bufs[:_RING]
        mbufs = bufs[_RING:2 * _RING]
        acc_sh = bufs[2 * _RING]
        lsems = bufs[2 * _RING + 1:3 * _RING + 1]
        ssems = bufs[3 * _RING + 1:4 * _RING + 1]

        cid = lax.axis_index("c")
        sid = lax.axis_index("s")
        wid = sid * NC + cid
        lbase = wid * EPW
        gbase = e0 + lbase

        def zrow(r, c):
            for j in range(D // 16):
                mbufs[0][r, pl.ds(j * 16, 16)] = jnp.zeros((16,), jnp.float32)
            return c

        lax.fori_loop(0, CH, zrow, 0)
        for t in range(RPT // CH):
            pltpu.sync_copy(mbufs[0],
                            acc_sh.at[pl.ds(sid * RPT + t * CH, CH)])
        plsc.subcore_barrier()

        def issue_load(c, b):
            pltpu.async_copy(m_hbm.at[pl.ds(lbase + c * CH, CH)],
                             mbufs[b], lsems[b])
            pltpu.async_copy(src_hbm.at[pl.ds(gbase + c * CH, CH)],
                             idxbs[b], lsems[b])

        def wait_load(b):
            pltpu.make_async_copy(m_hbm.at[pl.ds(0, CH)], mbufs[b],
                                  lsems[b]).wait()
            pltpu.make_async_copy(src_hbm.at[pl.ds(0, CH)], idxbs[b],
                                  lsems[b]).wait()

        def issue_scatter(b):
            pltpu.async_copy(mbufs[b], acc_sh.at[idxbs[b]], ssems[b],
                             add=True)

        def wait_scatter(b):
            pltpu.make_async_copy(mbufs[b], acc_sh.at[idxbs[b]],
                                  ssems[b]).wait()

        for b in range(_RING - 1):
            issue_load(b, b)

        def quad(i4, carry):
            for b in range(_RING):
                c = i4 * _RING + b
                tb = (b + 3) % _RING

                @pl.when(c >= 1)
                def _():
                    wait_scatter(tb)

                @pl.when(c + (_RING - 1) < NCHUNK)
                def _():
                    issue_load(c + (_RING - 1), tb)

                wait_load(b)
                issue_scatter(b)
            return carry

        lax.fori_loop(0, NCHUNK // _RING, quad, 0)

        wait_scatter(_RING - 1)
        wait_load(0)
        issue_scatter(0)
        wait_scatter(0)
        plsc.subcore_barrier()

        for t in range(RPT // CH):
            row0 = sid * RPT + t * CH
            pltpu.sync_copy(acc_sh.at[pl.ds(row0, CH)],
                            out_hbm.at[pl.ds(cid * NPAD + row0, CH)])

    return body


@functools.lru_cache(maxsize=None)
def _scatter_kernel(h):
    return pl.kernel(
        _make_scatter_body(h * EH),
        out_type=jax.ShapeDtypeStruct((NC * NPAD, D), jnp.float32),
        mesh=_mesh(),
        scratch_types=(
            [pltpu.VMEM((CH,), jnp.int32)] * _RING
            + [pltpu.VMEM((CH, D), jnp.float32)] * _RING
            + [pltpu.VMEM_SHARED((NPAD, D), jnp.float32)]
            + [pltpu.SemaphoreType.DMA] * (2 * _RING)
        ),
    )


def _final_body(mp0_ref, mp1_ref, x_ref, gam_ref, bet_ref, o_ref):
    msg = (mp0_ref[0:N, :] + mp0_ref[NPAD:NPAD + N, :]
           + mp1_ref[0:N, :] + mp1_ref[NPAD:NPAD + N, :])
    mean = jnp.mean(msg, axis=0, keepdims=True)
    var = jnp.mean((msg - mean) ** 2, axis=0, keepdims=True)
    norm = (msg - mean) / jnp.sqrt(var + 1e-5) * gam_ref[...] + bet_ref[...]
    o_ref[...] = _softplus(x_ref[...] + norm)


def _final_call(mp0, mp1, x, gamma, beta):
    return pl.pallas_call(
        _final_body,
        out_shape=jax.ShapeDtypeStruct((N, D), jnp.float32),
    )(mp0, mp1, x, gamma, beta)


def kernel(x, edge_attr, edge_source, edge_target, Wf, bf, Ws, bs, gamma, beta):
    src = edge_source.astype(jnp.int32)
    dst = edge_target.astype(jnp.int32)
    b2 = jnp.concatenate([bf, bs]).reshape(1, D2)
    wsrc = jnp.concatenate([Wf[:, :D].T, Ws[:, :D].T], axis=1)
    wdst = jnp.concatenate([Wf[:, D:2 * D].T, Ws[:, D:2 * D].T], axis=1)
    wea = jnp.concatenate([Wf[:, 2 * D:].T, Ws[:, 2 * D:].T], axis=1)

    p, q = _pq_call(x, wsrc, wdst, b2)
    gs0, gd0 = _gather_kernel(0)(p, q, src, dst)
    gs1, gd1 = _gather_kernel(1)(p, q, src, dst)
    m0 = _edge_call(edge_attr, gs0, gd0, wea, 0)
    m1 = _edge_call(edge_attr, gs1, gd1, wea, 1)
    mp0 = _scatter_kernel(0)(m0, src)
    mp1 = _scatter_kernel(1)(m1, src)
    return _final_call(mp0, mp1, x, gamma.reshape(1, D), beta.reshape(1, D))
